# Initial kernel scaffold; baseline (speedup 1.0000x reference)
import jax
import jax.numpy as jnp
from jax.experimental import pallas as pl


def kernel(query, value, reference_points, spatial_shapes, W_value, b_value, W_off, b_off, W_attn, b_attn, W_out, b_out):
    raise NotImplementedError("write your pallas kernel here")


# speedup vs baseline: 2008.0183x; 2008.0183x over previous
import functools

import jax
import jax.numpy as jnp
from jax import lax
from jax.experimental import pallas as pl
from jax.experimental.pallas import tpu as pltpu
from jax.experimental.pallas import tpu_sc as plsc

E = 256
HEADS = 8
POINTS = 4
HD = 32
H = 100
W = 100
LANES = HEADS * POINTS * 4

NW = 32
R = 20000
R_PAD = 20480
RPW = R_PAD // NW
BLK = 128
NBLK = RPW // BLK
CHUNK = 8


def _proj_body(x_ref, w_ref, b_ref, o_ref):
    o_ref[...] = (
        jnp.dot(x_ref[...], w_ref[...], preferred_element_type=jnp.float32)
        + b_ref[...]
    )


def _route_body(q_ref, rp_ref, wx_ref, bx_ref, wy_ref, by_ref, wl_ref,
                bl_ref, gg_ref, idx_ref, wgt_ref, *, rows_per_blk):
    qb = q_ref[...]
    X = (jnp.dot(qb, wx_ref[...], preferred_element_type=jnp.float32)
         + bx_ref[...] + (rp_ref[:, 0:1] * float(W) - 0.5))
    Y = (jnp.dot(qb, wy_ref[...], preferred_element_type=jnp.float32)
         + by_ref[...] + (rp_ref[:, 1:2] * float(H) - 0.5))
    Eo = jnp.exp(jnp.dot(qb, wl_ref[...], preferred_element_type=jnp.float32)
                 + bl_ref[...])
    Sden = jnp.dot(Eo, gg_ref[...], preferred_element_type=jnp.float32)
    AW = Eo / Sden
    L = lax.broadcasted_iota(jnp.int32, X.shape, 1)
    cx = L % 2
    cy = (L % 4) // 2
    x0 = jnp.floor(X)
    fx = X - x0
    y0 = jnp.floor(Y)
    fy = Y - y0
    wxv = jnp.where(cx == 1, fx, 1.0 - fx)
    wyv = jnp.where(cy == 1, fy, 1.0 - fy)
    xc = x0 + cx.astype(jnp.float32)
    yc = y0 + cy.astype(jnp.float32)
    valid = ((xc >= 0.0) & (xc <= float(W - 1))
             & (yc >= 0.0) & (yc <= float(H - 1)))
    wgt_ref[...] = jnp.where(valid, AW * wxv * wyv, 0.0)
    xi = jnp.clip(xc, 0.0, float(W - 1)).astype(jnp.int32)
    yi = jnp.clip(yc, 0.0, float(H - 1)).astype(jnp.int32)
    rglob = (lax.broadcasted_iota(jnp.int32, X.shape, 0)
             + pl.program_id(0) * rows_per_blk)
    b = rglob % 2
    hh = L // 16
    idx_ref[...] = (yi * W + xi) * (2 * HEADS) + b * HEADS + hh


def _out_body(a_ref, w_ref, b_ref, q_ref, o_ref):
    o_ref[...] = (
        jnp.dot(a_ref[...], w_ref[...], preferred_element_type=jnp.float32)
        + b_ref[...] + q_ref[...]
    )


def _accum_row(buf, r, wgt_v, rows_v, out_v):
    for h in range(HEADS):
        wv = wgt_v[r, h * 16:h * 16 + 16]
        a0 = jnp.zeros((16,), jnp.float32)
        a1 = jnp.zeros((16,), jnp.float32)
        for i in range(16):
            j = h * 16 + i
            wsc = wv[i]
            a0 = a0 + wsc * rows_v[buf, j, 0:16]
            a1 = a1 + wsc * rows_v[buf, j, 16:32]
        out_v[r, h * 32:h * 32 + 16] = a0
        out_v[r, h * 32 + 16:h * 32 + 32] = a1


def _sc_body(vt, idxp, wgtp, out, idx_v, wgt_v, rows_v, out_v, sem):
    wid = lax.axis_index("s") * 2 + lax.axis_index("c")
    base = wid * RPW

    def blk_body(blk, carry):
        b0 = base + blk * BLK
        pltpu.sync_copy(idxp.at[pl.ds(b0, BLK)], idx_v)
        pltpu.sync_copy(wgtp.at[pl.ds(b0, BLK)], wgt_v)

        def chunk_body(ck, carry2):
            r0 = ck * CHUNK
            copies = []
            for c in range(CHUNK):
                copies.append(
                    pltpu.async_copy(vt.at[idx_v.at[r0 + c]], rows_v.at[c],
                                     sem))
            for c in range(CHUNK):
                copies[c].wait()
                _accum_row(c, r0 + c, wgt_v, rows_v, out_v)
            return carry2

        lax.fori_loop(0, BLK // CHUNK, chunk_body, 0)
        pltpu.sync_copy(out_v, out.at[pl.ds(b0, BLK)])
        return carry

    lax.fori_loop(0, NBLK, blk_body, 0)


def _matmul_call(x, w, b, body, extra=(), rows_blk=400, out_cols=None):
    rows = x.shape[0]
    assert rows % rows_blk == 0
    nco = w.shape[1] if out_cols is None else out_cols
    in_specs = [
        pl.BlockSpec((rows_blk, x.shape[1]), lambda i: (i, 0)),
        pl.BlockSpec((w.shape[0], w.shape[1]), lambda i: (0, 0)),
        pl.BlockSpec((1, b.shape[1]), lambda i: (0, 0)),
    ]
    args = [x, w, b]
    for a in extra:
        in_specs.append(pl.BlockSpec((rows_blk, a.shape[1]), lambda i: (i, 0)))
        args.append(a)
    return pl.pallas_call(
        body,
        grid=(rows // rows_blk,),
        in_specs=in_specs,
        out_specs=pl.BlockSpec((rows_blk, nco), lambda i: (i, 0)),
        out_shape=jax.ShapeDtypeStruct((rows, nco), jnp.float32),
    )(*args)


def kernel(query, value, reference_points, spatial_shapes, W_value, b_value,
           W_off, b_off, W_attn, b_attn, W_out, b_out):
    NQ, BS, _ = query.shape
    NV = value.shape[0]

    q2d = query.reshape(NQ * BS, E)
    v2d = value.reshape(NV * BS, E)
    rp2 = jnp.transpose(reference_points[:, :, 0, :], (1, 0, 2)).reshape(
        NQ * BS, 2)

    q2dp = jnp.pad(q2d, ((0, R_PAD - R), (0, 0)))
    rp2p = jnp.pad(rp2, ((0, R_PAD - R), (0, 0)))

    Wo3 = W_off.reshape(E, HEADS * POINTS, 2)
    Wx = jnp.repeat(Wo3[:, :, 0], 4, axis=1)
    Wy = jnp.repeat(Wo3[:, :, 1], 4, axis=1)
    bo2 = b_off.reshape(HEADS * POINTS, 2)
    bx = jnp.repeat(bo2[:, 0], 4).reshape(1, LANES)
    by = jnp.repeat(bo2[:, 1], 4).reshape(1, LANES)
    Wl = jnp.repeat(W_attn, 4, axis=1)
    bl = jnp.repeat(b_attn, 4).reshape(1, LANES)
    gidx = jnp.arange(LANES) // 16
    GG = 0.25 * (gidx[:, None] == gidx[None, :]).astype(jnp.float32)

    vt = _matmul_call(v2d, W_value, b_value.reshape(1, E), _proj_body)
    vt_rows = vt.reshape(NV * BS * HEADS, HD)

    ROWS_BLK = 512
    grid = R_PAD // ROWS_BLK
    idxp, wgtp = pl.pallas_call(
        functools.partial(_route_body, rows_per_blk=ROWS_BLK),
        grid=(grid,),
        in_specs=[
            pl.BlockSpec((ROWS_BLK, E), lambda i: (i, 0)),
            pl.BlockSpec((ROWS_BLK, 2), lambda i: (i, 0)),
            pl.BlockSpec((E, LANES), lambda i: (0, 0)),
            pl.BlockSpec((1, LANES), lambda i: (0, 0)),
            pl.BlockSpec((E, LANES), lambda i: (0, 0)),
            pl.BlockSpec((1, LANES), lambda i: (0, 0)),
            pl.BlockSpec((E, LANES), lambda i: (0, 0)),
            pl.BlockSpec((1, LANES), lambda i: (0, 0)),
            pl.BlockSpec((LANES, LANES), lambda i: (0, 0)),
        ],
        out_specs=[
            pl.BlockSpec((ROWS_BLK, LANES), lambda i: (i, 0)),
            pl.BlockSpec((ROWS_BLK, LANES), lambda i: (i, 0)),
        ],
        out_shape=[
            jax.ShapeDtypeStruct((R_PAD, LANES), jnp.int32),
            jax.ShapeDtypeStruct((R_PAD, LANES), jnp.float32),
        ],
    )(q2dp, rp2p, Wx, bx, Wy, by, Wl, bl, GG)

    mesh = plsc.VectorSubcoreMesh(core_axis_name="c", subcore_axis_name="s",
                                  num_cores=2, num_subcores=16)
    attn = pl.kernel(
        _sc_body,
        mesh=mesh,
        out_type=jax.ShapeDtypeStruct((R_PAD, E), jnp.float32),
        scratch_types=[
            pltpu.VMEM((BLK, LANES), jnp.int32),
            pltpu.VMEM((BLK, LANES), jnp.float32),
            pltpu.VMEM((CHUNK, LANES, HD), jnp.float32),
            pltpu.VMEM((BLK, E), jnp.float32),
            pltpu.SemaphoreType.DMA,
        ],
        compiler_params=pltpu.CompilerParams(use_tc_tiling_on_sc=False),
    )(vt_rows, idxp, wgtp)

    out2d = _matmul_call(attn, W_out, b_out.reshape(1, E), _out_body,
                         extra=(q2dp,), rows_blk=512)
    return out2d[:R].reshape(NQ, BS, E)


# speedup vs baseline: 3235.4534x; 1.6113x over previous
import jax
import jax.numpy as jnp
from jax import lax
from jax.experimental import pallas as pl
from jax.experimental.pallas import tpu as pltpu
from jax.experimental.pallas import tpu_sc as plsc

E = 256
HEADS = 8
POINTS = 4
HD = 32
H = 100
W = 100
NQ = 10000
BS = 2
LANES = 128
NP = HEADS * POINTS

R = NQ * BS
NWORK = 32
RPW = R // NWORK
BLK = 125
NBLK = RPW // BLK
CHUNK = 5
NCH = BLK // CHUNK


def _k1_body(v_ref, w_ref, bv_ref, o_ref, scr):
    b = pl.program_id(0)
    a = (jnp.dot(v_ref[:, b, :], w_ref[0],
                 preferred_element_type=jnp.float32) + bv_ref[0])
    scr[pl.ds(0, NQ), :] = a
    scr[pl.ds(NQ, 104), :] = jnp.zeros((104, HD), jnp.float32)
    o_ref[0] = jnp.concatenate(
        [scr[pl.ds(k, NQ), :] for k in (0, 1, W, W + 1)], axis=1)


def _k2_body(q_ref, rp_ref, wx_ref, bx_ref, wy_ref, by_ref, wl_ref, bl_ref,
             wx32_ref, bx32_ref, wy32_ref, by32_ref, gg_ref, expm_ref,
             idx_ref, wgt_ref):
    b = pl.program_id(1)
    qb = q_ref[:, b, :]
    rpx = rp_ref[0][:, 0:1] * float(W) - 0.5
    rpy = rp_ref[0][:, 1:2] * float(H) - 0.5
    X = jnp.dot(qb, wx_ref[...], preferred_element_type=jnp.float32) \
        + bx_ref[...] + rpx
    Y = jnp.dot(qb, wy_ref[...], preferred_element_type=jnp.float32) \
        + by_ref[...] + rpy
    Eo = jnp.exp(jnp.dot(qb, wl_ref[...], preferred_element_type=jnp.float32)
                 + bl_ref[...])
    Sden = jnp.dot(Eo, gg_ref[...], preferred_element_type=jnp.float32)
    AW = Eo / Sden
    X32 = jnp.dot(qb, wx32_ref[...], preferred_element_type=jnp.float32) \
        + bx32_ref[...] + rpx
    Y32 = jnp.dot(qb, wy32_ref[...], preferred_element_type=jnp.float32) \
        + by32_ref[...] + rpy
    xs32 = jnp.clip(jnp.floor(X32), 0.0, float(W - 2))
    ys32 = jnp.clip(jnp.floor(Y32), 0.0, float(H - 2))
    l32 = lax.broadcasted_iota(jnp.int32, X32.shape, 1)
    plane = b * HEADS + l32 // POINTS
    idx_ref[0, :, 0:NP] = (plane * (H * W) + ys32.astype(jnp.int32) * W
                           + xs32.astype(jnp.int32))
    xs128 = jnp.dot(xs32, expm_ref[...], preferred_element_type=jnp.float32)
    ys128 = jnp.dot(ys32, expm_ref[...], preferred_element_type=jnp.float32)
    l = lax.broadcasted_iota(jnp.int32, X.shape, 1)
    kx = (l % 2).astype(jnp.float32)
    ky = ((l % 4) // 2).astype(jnp.float32)
    tentx = jnp.maximum(0.0, 1.0 - jnp.abs(X - (xs128 + kx)))
    tenty = jnp.maximum(0.0, 1.0 - jnp.abs(Y - (ys128 + ky)))
    wgt_ref[0] = AW * tentx * tenty


def _k4_body(a_ref, w_ref, b_ref, q_ref, o_ref):
    b = pl.program_id(1)
    o_ref[:, b, :] = (
        jnp.dot(a_ref[0], w_ref[...], preferred_element_type=jnp.float32)
        + b_ref[...] + q_ref[:, b, :])


def _accum_row(buf, r, wgt_v, rows_v, out_v):
    def hbody(h, carry):
        w16 = wgt_v[r, pl.ds(h * 16, 16)]
        a0 = jnp.zeros((16,), jnp.float32)
        a1 = jnp.zeros((16,), jnp.float32)
        for p in range(POINTS):
            j = h * POINTS + p
            for c in range(4):
                wsc = w16[p * 4 + c]
                a0 = a0 + wsc * rows_v[buf, j, pl.ds(c * 32, 16)]
                a1 = a1 + wsc * rows_v[buf, j, pl.ds(c * 32 + 16, 16)]
        out_v[r, pl.ds(h * 32, 16)] = a0
        out_v[r, pl.ds(h * 32 + 16, 16)] = a1
        return carry
    lax.fori_loop(0, HEADS, hbody, 0)


def _sc_body(vtp, idxp, wgtp, out, idx_v, wgt_v, rows_v, out_v, sem):
    wid = lax.axis_index("s") * 2 + lax.axis_index("c")
    base = wid * RPW

    def fire(ck, half):
        r0 = ck * CHUNK
        for c in range(CHUNK):
            pltpu.async_copy(
                vtp.at[idx_v.at[r0 + c, pl.ds(0, NP)]],
                rows_v.at[half * CHUNK + c], sem)

    def drain(ck, half):
        r0 = ck * CHUNK
        for c in range(CHUNK):
            pltpu.make_async_copy(
                vtp.at[idx_v.at[r0 + c, pl.ds(0, NP)]],
                rows_v.at[half * CHUNK + c], sem).wait()
            _accum_row(half * CHUNK + c, r0 + c, wgt_v, rows_v, out_v)

    def blk_body(blk, carry):
        b0 = base + blk * BLK
        pltpu.sync_copy(idxp.at[pl.ds(b0, BLK)], idx_v)
        pltpu.sync_copy(wgtp.at[pl.ds(b0, BLK)], wgt_v)
        fire(0, 0)

        def pair_body(jp, carry2):
            fire(2 * jp + 1, 1)
            drain(2 * jp, 0)
            fire(2 * jp + 2, 0)
            drain(2 * jp + 1, 1)
            return carry2

        lax.fori_loop(0, (NCH - 1) // 2, pair_body, 0)
        drain(NCH - 1, 0)
        pltpu.sync_copy(out_v, out.at[pl.ds(b0, BLK)])
        return carry

    lax.fori_loop(0, NBLK, blk_body, 0)


def kernel(query, value, reference_points, spatial_shapes, W_value, b_value,
           W_off, b_off, W_attn, b_attn, W_out, b_out):
    f32 = jnp.float32

    Wv3 = W_value.reshape(E, HEADS, HD).transpose(1, 0, 2)
    bv3 = b_value.reshape(HEADS, 1, HD)
    Wo3 = W_off.reshape(E, NP, 2)
    Wx32 = Wo3[:, :, 0]
    Wy32 = Wo3[:, :, 1]
    bo2 = b_off.reshape(NP, 2)
    bx32 = bo2[:, 0].reshape(1, NP)
    by32 = bo2[:, 1].reshape(1, NP)
    Wx = jnp.repeat(Wx32, 4, axis=1)
    Wy = jnp.repeat(Wy32, 4, axis=1)
    bx = jnp.repeat(bx32[0], 4).reshape(1, LANES)
    by = jnp.repeat(by32[0], 4).reshape(1, LANES)
    Wl = jnp.repeat(W_attn, 4, axis=1)
    bl = jnp.repeat(b_attn, 4).reshape(1, LANES)
    gidx = jnp.arange(LANES) // 16
    GG = 0.25 * (gidx[:, None] == gidx[None, :]).astype(f32)
    EXPM = (jnp.arange(NP)[:, None] == (jnp.arange(LANES)[None, :] // 4)
            ).astype(f32)
    rp3 = reference_points.reshape(BS, NQ, 2)

    vtp = pl.pallas_call(
        _k1_body,
        grid=(BS, HEADS),
        in_specs=[
            pl.BlockSpec((NQ, BS, E), lambda b, h: (0, 0, 0)),
            pl.BlockSpec((1, E, HD), lambda b, h: (h, 0, 0)),
            pl.BlockSpec((1, 1, HD), lambda b, h: (h, 0, 0)),
        ],
        out_specs=pl.BlockSpec((1, NQ, LANES), lambda b, h: (b * HEADS + h, 0, 0)),
        out_shape=jax.ShapeDtypeStruct((BS * HEADS, NQ, LANES), f32),
        scratch_shapes=[pltpu.VMEM((NQ + 104, HD), f32)],
        compiler_params=pltpu.CompilerParams(
            vmem_limit_bytes=100 * 1024 * 1024),
    )(value, Wv3, bv3)
    vtp_flat = vtp.reshape(BS * HEADS * NQ, LANES)

    QB = 200
    gq = NQ // QB
    idxp, wgtp = pl.pallas_call(
        _k2_body,
        grid=(gq, BS),
        in_specs=[
            pl.BlockSpec((QB, BS, E), lambda i, b: (i, 0, 0)),
            pl.BlockSpec((1, QB, 2), lambda i, b: (b, i, 0)),
            pl.BlockSpec((E, LANES), lambda i, b: (0, 0)),
            pl.BlockSpec((1, LANES), lambda i, b: (0, 0)),
            pl.BlockSpec((E, LANES), lambda i, b: (0, 0)),
            pl.BlockSpec((1, LANES), lambda i, b: (0, 0)),
            pl.BlockSpec((E, LANES), lambda i, b: (0, 0)),
            pl.BlockSpec((1, LANES), lambda i, b: (0, 0)),
            pl.BlockSpec((E, NP), lambda i, b: (0, 0)),
            pl.BlockSpec((1, NP), lambda i, b: (0, 0)),
            pl.BlockSpec((E, NP), lambda i, b: (0, 0)),
            pl.BlockSpec((1, NP), lambda i, b: (0, 0)),
            pl.BlockSpec((LANES, LANES), lambda i, b: (0, 0)),
            pl.BlockSpec((NP, LANES), lambda i, b: (0, 0)),
        ],
        out_specs=[
            pl.BlockSpec((1, QB, LANES), lambda i, b: (b, i, 0)),
            pl.BlockSpec((1, QB, LANES), lambda i, b: (b, i, 0)),
        ],
        out_shape=[
            jax.ShapeDtypeStruct((BS, NQ, LANES), jnp.int32),
            jax.ShapeDtypeStruct((BS, NQ, LANES), f32),
        ],
    )(query, rp3, Wx, bx, Wy, by, Wl, bl, Wx32, bx32, Wy32, by32, GG, EXPM)
    idx2d = idxp.reshape(R, LANES)
    wgt2d = wgtp.reshape(R, LANES)

    mesh = plsc.VectorSubcoreMesh(core_axis_name="c", subcore_axis_name="s",
                                  num_cores=2, num_subcores=16)
    attn = pl.kernel(
        _sc_body,
        mesh=mesh,
        out_type=jax.ShapeDtypeStruct((R, E), f32),
        scratch_types=[
            pltpu.VMEM((BLK, LANES), jnp.int32),
            pltpu.VMEM((BLK, LANES), f32),
            pltpu.VMEM((2 * CHUNK, NP, LANES), f32),
            pltpu.VMEM((BLK, E), f32),
            pltpu.SemaphoreType.DMA,
        ],
        compiler_params=pltpu.CompilerParams(use_tc_tiling_on_sc=False),
    )(vtp_flat, idx2d, wgt2d)
    attn3 = attn.reshape(BS, NQ, E)

    MB = 400
    gm = NQ // MB
    out = pl.pallas_call(
        _k4_body,
        grid=(gm, BS),
        in_specs=[
            pl.BlockSpec((1, MB, E), lambda i, b: (b, i, 0)),
            pl.BlockSpec((E, E), lambda i, b: (0, 0)),
            pl.BlockSpec((1, E), lambda i, b: (0, 0)),
            pl.BlockSpec((MB, BS, E), lambda i, b: (i, 0, 0)),
        ],
        out_specs=pl.BlockSpec((MB, BS, E), lambda i, b: (i, 0, 0)),
        out_shape=jax.ShapeDtypeStruct((NQ, BS, E), f32),
    )(attn3, W_out, b_out.reshape(1, E), query)
    return out


# speedup vs baseline: 4150.7651x; 1.2829x over previous
import jax
import jax.numpy as jnp
from jax import lax
from jax.experimental import pallas as pl
from jax.experimental.pallas import tpu as pltpu
from jax.experimental.pallas import tpu_sc as plsc

E = 256
HEADS = 8
POINTS = 4
HD = 32
H = 100
W = 100
NQ = 10000
BS = 2
LANES = 128
NP = HEADS * POINTS

R = NQ * BS
NWORK = 32
RPW = R // NWORK
BLK = 125
NBLK = RPW // BLK
CHUNK = 5
NCH = BLK // CHUNK


def _k1_body(v_ref, w_ref, bv_ref, o_ref, scr, scr_v):
    b = pl.program_id(0)
    h = pl.program_id(1)

    @pl.when(h == 0)
    def _stage():
        scr_v[...] = v_ref[:, b, :]

    a = (jnp.dot(scr_v[...], w_ref[0],
                 preferred_element_type=jnp.float32) + bv_ref[0])
    scr[pl.ds(0, NQ), :] = a
    o_ref[0] = jnp.concatenate(
        [scr[pl.ds(k, NQ), :] for k in (0, 1, W, W + 1)], axis=1)


def _k2_body(q_ref, rp_ref, wxyl_ref, bxyl_ref, wx32_ref, bx32_ref,
             wy32_ref, by32_ref, gg_ref, expm_ref, idx_ref, wgt_ref):
    b = pl.program_id(1)
    qb = q_ref[:, b, :]
    rpx = rp_ref[0][:, 0:1] * float(W) - 0.5
    rpy = rp_ref[0][:, 1:2] * float(H) - 0.5
    XYL = jnp.dot(qb, wxyl_ref[...], preferred_element_type=jnp.float32) \
        + bxyl_ref[...]
    X = XYL[:, 0:LANES] + rpx
    Y = XYL[:, LANES:2 * LANES] + rpy
    Eo = jnp.exp(XYL[:, 2 * LANES:3 * LANES])
    Sden = jnp.dot(Eo, gg_ref[...], preferred_element_type=jnp.float32)
    AW = Eo / Sden
    X32 = jnp.dot(qb, wx32_ref[...], preferred_element_type=jnp.float32) \
        + bx32_ref[...] + rpx
    Y32 = jnp.dot(qb, wy32_ref[...], preferred_element_type=jnp.float32) \
        + by32_ref[...] + rpy
    xs32 = jnp.clip(jnp.floor(X32), 0.0, float(W - 2))
    ys32 = jnp.clip(jnp.floor(Y32), 0.0, float(H - 2))
    l32 = lax.broadcasted_iota(jnp.int32, X32.shape, 1)
    plane = b * HEADS + l32 // POINTS
    idx_ref[0, :, 0:NP] = (plane * (H * W) + ys32.astype(jnp.int32) * W
                           + xs32.astype(jnp.int32))
    xs128 = jnp.dot(xs32, expm_ref[...], preferred_element_type=jnp.float32)
    ys128 = jnp.dot(ys32, expm_ref[...], preferred_element_type=jnp.float32)
    l = lax.broadcasted_iota(jnp.int32, X.shape, 1)
    kx = (l % 2).astype(jnp.float32)
    ky = ((l % 4) // 2).astype(jnp.float32)
    tentx = jnp.maximum(0.0, 1.0 - jnp.abs(X - (xs128 + kx)))
    tenty = jnp.maximum(0.0, 1.0 - jnp.abs(Y - (ys128 + ky)))
    wgt_ref[0] = AW * tentx * tenty


def _k4_body(a_ref, w_ref, b_ref, q_ref, o_ref):
    b = pl.program_id(1)
    o_ref[:, b, :] = (
        jnp.dot(a_ref[0], w_ref[...], preferred_element_type=jnp.float32)
        + b_ref[...] + q_ref[:, b, :])


def _accum_row(buf, r, wgt_v, rows_v, out_v):
    def hbody(h, carry):
        w16 = wgt_v[r, pl.ds(h * 16, 16)]
        a0 = jnp.zeros((16,), jnp.float32)
        a1 = jnp.zeros((16,), jnp.float32)
        for p in range(POINTS):
            j = h * POINTS + p
            for c in range(4):
                wsc = w16[p * 4 + c]
                a0 = a0 + wsc * rows_v[buf, j, pl.ds(c * 32, 16)]
                a1 = a1 + wsc * rows_v[buf, j, pl.ds(c * 32 + 16, 16)]
        out_v[r, pl.ds(h * 32, 16)] = a0
        out_v[r, pl.ds(h * 32 + 16, 16)] = a1
        return carry
    lax.fori_loop(0, HEADS, hbody, 0)


def _sc_body(vtp, idxp, wgtp, out, idx_v, wgt_v, rows_v, out_v, sem):
    wid = lax.axis_index("s") * 2 + lax.axis_index("c")
    base = wid * RPW

    def fire(ck, half):
        r0 = ck * CHUNK
        for c in range(CHUNK):
            pltpu.async_copy(
                vtp.at[idx_v.at[r0 + c, pl.ds(0, NP)]],
                rows_v.at[half * CHUNK + c], sem)

    def drain(ck, half):
        r0 = ck * CHUNK
        for c in range(CHUNK):
            pltpu.make_async_copy(
                vtp.at[idx_v.at[r0 + c, pl.ds(0, NP)]],
                rows_v.at[half * CHUNK + c], sem).wait()
            _accum_row(half * CHUNK + c, r0 + c, wgt_v, rows_v, out_v)

    def blk_body(blk, carry):
        b0 = base + blk * BLK
        pltpu.sync_copy(idxp.at[pl.ds(b0, BLK)], idx_v)
        pltpu.sync_copy(wgtp.at[pl.ds(b0, BLK)], wgt_v)
        fire(0, 0)

        def pair_body(jp, carry2):
            fire(2 * jp + 1, 1)
            drain(2 * jp, 0)
            fire(2 * jp + 2, 0)
            drain(2 * jp + 1, 1)
            return carry2

        lax.fori_loop(0, (NCH - 1) // 2, pair_body, 0)
        drain(NCH - 1, 0)
        pltpu.sync_copy(out_v, out.at[pl.ds(b0, BLK)])
        return carry

    lax.fori_loop(0, NBLK, blk_body, 0)


def kernel(query, value, reference_points, spatial_shapes, W_value, b_value,
           W_off, b_off, W_attn, b_attn, W_out, b_out):
    f32 = jnp.float32

    Wv3 = W_value.reshape(E, HEADS, HD).transpose(1, 0, 2)
    bv3 = b_value.reshape(HEADS, 1, HD)
    Wo3 = W_off.reshape(E, NP, 2)
    Wx32 = Wo3[:, :, 0]
    Wy32 = Wo3[:, :, 1]
    bo2 = b_off.reshape(NP, 2)
    bx32 = bo2[:, 0].reshape(1, NP)
    by32 = bo2[:, 1].reshape(1, NP)
    Wx = jnp.repeat(Wx32, 4, axis=1)
    Wy = jnp.repeat(Wy32, 4, axis=1)
    bx = jnp.repeat(bx32[0], 4).reshape(1, LANES)
    by = jnp.repeat(by32[0], 4).reshape(1, LANES)
    Wl = jnp.repeat(W_attn, 4, axis=1)
    bl = jnp.repeat(b_attn, 4).reshape(1, LANES)
    WXYL = jnp.concatenate([Wx, Wy, Wl], axis=1)
    bXYL = jnp.concatenate([bx, by, bl], axis=1)
    gidx = jnp.arange(LANES) // 16
    GG = 0.25 * (gidx[:, None] == gidx[None, :]).astype(f32)
    EXPM = (jnp.arange(NP)[:, None] == (jnp.arange(LANES)[None, :] // 4)
            ).astype(f32)
    rp3 = reference_points.reshape(BS, NQ, 2)

    vtp = pl.pallas_call(
        _k1_body,
        grid=(BS, HEADS),
        in_specs=[
            pl.BlockSpec((NQ, BS, E), lambda b, h: (0, 0, 0)),
            pl.BlockSpec((1, E, HD), lambda b, h: (h, 0, 0)),
            pl.BlockSpec((1, 1, HD), lambda b, h: (h, 0, 0)),
        ],
        out_specs=pl.BlockSpec((1, NQ, LANES), lambda b, h: (b * HEADS + h, 0, 0)),
        out_shape=jax.ShapeDtypeStruct((BS * HEADS, NQ, LANES), f32),
        scratch_shapes=[pltpu.VMEM((NQ + 104, HD), f32),
                        pltpu.VMEM((NQ, E), f32)],
        compiler_params=pltpu.CompilerParams(
            vmem_limit_bytes=100 * 1024 * 1024),
    )(value, Wv3, bv3)
    vtp_flat = vtp.reshape(BS * HEADS * NQ, LANES)

    QB = 1000
    gq = NQ // QB
    idxp, wgtp = pl.pallas_call(
        _k2_body,
        grid=(gq, BS),
        in_specs=[
            pl.BlockSpec((QB, BS, E), lambda i, b: (i, 0, 0)),
            pl.BlockSpec((1, QB, 2), lambda i, b: (b, i, 0)),
            pl.BlockSpec((E, 3 * LANES), lambda i, b: (0, 0)),
            pl.BlockSpec((1, 3 * LANES), lambda i, b: (0, 0)),
            pl.BlockSpec((E, NP), lambda i, b: (0, 0)),
            pl.BlockSpec((1, NP), lambda i, b: (0, 0)),
            pl.BlockSpec((E, NP), lambda i, b: (0, 0)),
            pl.BlockSpec((1, NP), lambda i, b: (0, 0)),
            pl.BlockSpec((LANES, LANES), lambda i, b: (0, 0)),
            pl.BlockSpec((NP, LANES), lambda i, b: (0, 0)),
        ],
        out_specs=[
            pl.BlockSpec((1, QB, LANES), lambda i, b: (b, i, 0)),
            pl.BlockSpec((1, QB, LANES), lambda i, b: (b, i, 0)),
        ],
        out_shape=[
            jax.ShapeDtypeStruct((BS, NQ, LANES), jnp.int32),
            jax.ShapeDtypeStruct((BS, NQ, LANES), f32),
        ],
    )(query, rp3, WXYL, bXYL, Wx32, bx32, Wy32, by32, GG, EXPM)
    idx2d = idxp.reshape(R, LANES)
    wgt2d = wgtp.reshape(R, LANES)

    mesh = plsc.VectorSubcoreMesh(core_axis_name="c", subcore_axis_name="s",
                                  num_cores=2, num_subcores=16)
    attn = pl.kernel(
        _sc_body,
        mesh=mesh,
        out_type=jax.ShapeDtypeStruct((R, E), f32),
        scratch_types=[
            pltpu.VMEM((BLK, LANES), jnp.int32),
            pltpu.VMEM((BLK, LANES), f32),
            pltpu.VMEM((2 * CHUNK, NP, LANES), f32),
            pltpu.VMEM((BLK, E), f32),
            pltpu.SemaphoreType.DMA,
        ],
        compiler_params=pltpu.CompilerParams(use_tc_tiling_on_sc=False),
    )(vtp_flat, idx2d, wgt2d)
    attn3 = attn.reshape(BS, NQ, E)

    MB = 1000
    gm = NQ // MB
    out = pl.pallas_call(
        _k4_body,
        grid=(gm, BS),
        in_specs=[
            pl.BlockSpec((1, MB, E), lambda i, b: (b, i, 0)),
            pl.BlockSpec((E, E), lambda i, b: (0, 0)),
            pl.BlockSpec((1, E), lambda i, b: (0, 0)),
            pl.BlockSpec((MB, BS, E), lambda i, b: (i, 0, 0)),
        ],
        out_specs=pl.BlockSpec((MB, BS, E), lambda i, b: (i, 0, 0)),
        out_shape=jax.ShapeDtypeStruct((NQ, BS, E), f32),
    )(attn3, W_out, b_out.reshape(1, E), query)
    return out
